# baseline (speedup 1.0000x reference)
import functools

import jax
import jax.numpy as jnp
from jax import lax
from jax.experimental import pallas as pl
from jax.experimental.pallas import tpu as pltpu
from jax.experimental.pallas import tpu_sc as plsc

NC = 2
NS = 16
NW = NC * NS
LANES = 16

K = 26
TPAD = 688


def _sc_call(nt, nbt, dt4, et4, la, lb):
    mesh = plsc.VectorSubcoreMesh(
        core_axis_name="c", subcore_axis_name="s", num_cores=NC, num_subcores=NS
    )

    @functools.partial(
        pl.kernel,
        mesh=mesh,
        out_type=jax.ShapeDtypeStruct((K, nt, nbt, 8, 128), jnp.float32),
        scratch_types=[
            pltpu.VMEM((TPAD,), jnp.float32),
            pltpu.VMEM((TPAD,), jnp.float32),
            pltpu.VMEM((nt, 8, 128), jnp.float32),
            pltpu.VMEM((nt, 8, 128), jnp.int32),
            pltpu.VMEM((K, 8, 128), jnp.float32),
            pltpu.VMEM((K, 8, 128), jnp.float32),
            pltpu.SemaphoreType.DMA,
            pltpu.SemaphoreType.DMA,
        ],
        compiler_params=pltpu.CompilerParams(needs_layout_passes=False),
    )
    def run(dt_hbm, et_hbm, la_hbm, lb_hbm, out_hbm,
            tbl_a, tbl_nb, dt_v, et_v, ob0, ob1, sem0, sem1):
        wid = lax.axis_index("s") * NC + lax.axis_index("c")
        pltpu.sync_copy(dt_hbm.at[:, wid], dt_v)
        pltpu.sync_copy(et_hbm.at[:, wid], et_v)
        pltpu.sync_copy(la_hbm, tbl_a)
        pltpu.sync_copy(lb_hbm, tbl_nb)
        for t in range(TPAD // LANES):
            s = pl.ds(t * LANES, LANES)
            tbl_a[s] = jnp.exp(tbl_a[s])
            tbl_nb[s] = -jnp.exp(tbl_nb[s])

        bufs = (ob0, ob1)
        sems = (sem0, sem1)

        def compute(t, ob):
            @plsc.parallel_loop(0, 8)
            def vstep(sp):
                s = sp
                lanes = [pl.ds(j * LANES, LANES) for j in range(8)]
                dts = [dt_v[t, s, ln] for ln in lanes]
                tis = [et_v[t, s, ln] * K for ln in lanes]

                @plsc.parallel_loop(0, K, unroll=2)
                def kstep(k):
                    for ln, dtv, ti in zip(lanes, dts, tis):
                        a = plsc.load_gather(tbl_a, [ti + k])
                        nb = plsc.load_gather(tbl_nb, [ti + k])
                        ob[k, s, ln] = a * jnp.exp(nb * dtv)

        def start_out(t, b):
            pltpu.async_copy(bufs[b], out_hbm.at[:, t, wid], sems[b])

        def wait_out(b):
            pltpu.make_async_copy(bufs[b], out_hbm.at[:, 0, wid], sems[b]).wait()

        def pair(i, carry):
            t0 = 2 * i

            @pl.when(i > 0)
            def _():
                wait_out(0)

            compute(t0, ob0)
            start_out(t0, 0)

            @pl.when(i > 0)
            def _():
                wait_out(1)

            compute(t0 + 1, ob1)
            start_out(t0 + 1, 1)
            return carry

        lax.fori_loop(0, nt // 2, pair, 0)
        wait_out(0)
        compute(nt - 1, ob0)
        start_out(nt - 1, 0)
        wait_out(0)
        wait_out(1)

    return run(dt4, et4, la, lb)


def kernel(dt, event_types, log_alpha, log_beta):
    batch, nev = dt.shape
    nt = nev // 8
    nbt = batch // 128
    dt4 = dt.reshape(nbt, 128, nt, 8).transpose(2, 0, 3, 1)
    et4 = (
        event_types.astype(jnp.int32).reshape(nbt, 128, nt, 8).transpose(2, 0, 3, 1)
    )
    pad = TPAD - K * K
    la = jnp.pad(log_alpha.reshape(-1), (0, pad))
    lb = jnp.pad(log_beta.reshape(-1), (0, pad))
    arr = _sc_call(nt, nbt, dt4, et4, la, lb)
    return arr.transpose(2, 4, 1, 3, 0).reshape(batch, nev, K)


# speedup vs baseline: 1.0700x; 1.0700x over previous
import functools

import jax
import jax.numpy as jnp
from jax import lax
from jax.experimental import pallas as pl
from jax.experimental.pallas import tpu as pltpu
from jax.experimental.pallas import tpu_sc as plsc

NC = 2
NS = 16
NW = NC * NS
LANES = 16

K = 26
TPAD = 688


def _sc_call(nt, nbt, dt4, et4, la, lb):
    mesh = plsc.VectorSubcoreMesh(
        core_axis_name="c", subcore_axis_name="s", num_cores=NC, num_subcores=NS
    )

    @functools.partial(
        pl.kernel,
        mesh=mesh,
        out_type=jax.ShapeDtypeStruct((K, nt, nbt, 8, 128), jnp.float32),
        scratch_types=[
            pltpu.VMEM((TPAD,), jnp.float32),
            pltpu.VMEM((TPAD,), jnp.float32),
            pltpu.VMEM((nt, 8, 128), jnp.float32),
            pltpu.VMEM((nt, 8, 128), jnp.int32),
            pltpu.VMEM((K, 8, 128), jnp.float32),
            pltpu.VMEM((K, 8, 128), jnp.float32),
            pltpu.SemaphoreType.DMA,
            pltpu.SemaphoreType.DMA,
        ],
        compiler_params=pltpu.CompilerParams(needs_layout_passes=False),
    )
    def run(dt_hbm, et_hbm, la_hbm, lb_hbm, out_hbm,
            tbl_a, tbl_nb, dt_v, et_v, ob0, ob1, sem0, sem1):
        wid = lax.axis_index("s") * NC + lax.axis_index("c")
        pltpu.sync_copy(dt_hbm.at[:, wid], dt_v)
        pltpu.sync_copy(et_hbm.at[:, wid], et_v)
        pltpu.sync_copy(la_hbm, tbl_a)
        pltpu.sync_copy(lb_hbm, tbl_nb)
        for t in range(TPAD // LANES):
            s = pl.ds(t * LANES, LANES)
            tbl_a[s] = jnp.exp(tbl_a[s])
            tbl_nb[s] = -jnp.exp(tbl_nb[s])

        bufs = (ob0, ob1)
        sems = (sem0, sem1)

        def compute(t, ob):
            @plsc.parallel_loop(0, 16)
            def vstep(sp):
                s = sp >> 1
                lq = sp & 1
                lanes = [pl.ds(lq * (4 * LANES) + j * LANES, LANES) for j in range(4)]
                dts = [dt_v[t, s, ln] for ln in lanes]
                tis = [et_v[t, s, ln] * K for ln in lanes]

                @plsc.parallel_loop(0, K, unroll=2)
                def kstep(k):
                    for ln, dtv, ti in zip(lanes, dts, tis):
                        a = plsc.load_gather(tbl_a, [ti + k])
                        nb = plsc.load_gather(tbl_nb, [ti + k])
                        ob[k, s, ln] = a * jnp.exp(nb * dtv)

        def start_out(t, b):
            pltpu.async_copy(bufs[b], out_hbm.at[:, t, wid], sems[b])

        def wait_out(b):
            pltpu.make_async_copy(bufs[b], out_hbm.at[:, 0, wid], sems[b]).wait()

        def pair(i, carry):
            t0 = 2 * i

            @pl.when(i > 0)
            def _():
                wait_out(0)

            compute(t0, ob0)
            start_out(t0, 0)

            @pl.when(i > 0)
            def _():
                wait_out(1)

            compute(t0 + 1, ob1)
            start_out(t0 + 1, 1)
            return carry

        lax.fori_loop(0, nt // 2, pair, 0)
        wait_out(0)
        compute(nt - 1, ob0)
        start_out(nt - 1, 0)
        wait_out(0)
        wait_out(1)

    return run(dt4, et4, la, lb)


def kernel(dt, event_types, log_alpha, log_beta):
    batch, nev = dt.shape
    nt = nev // 8
    nbt = batch // 128
    dt4 = dt.reshape(nbt, 128, nt, 8).transpose(2, 0, 3, 1)
    et4 = (
        event_types.astype(jnp.int32).reshape(nbt, 128, nt, 8).transpose(2, 0, 3, 1)
    )
    pad = TPAD - K * K
    la = jnp.pad(log_alpha.reshape(-1), (0, pad))
    lb = jnp.pad(log_beta.reshape(-1), (0, pad))
    arr = _sc_call(nt, nbt, dt4, et4, la, lb)
    return arr.transpose(2, 4, 1, 3, 0).reshape(batch, nev, K)
